# baseline (speedup 1.0000x reference)
import functools

import jax
import jax.numpy as jnp
from jax import lax
from jax.experimental import pallas as pl
from jax.experimental.pallas import tpu as pltpu
from jax.experimental.pallas import tpu_sc as plsc

N_NODES = 10000
N_EDGES = 320000
D_FEAT = 128

NC = 2
NS = 16
L = 16
NW = NC * NS
EPW = N_EDGES // NW
C = 80
NCHUNK = EPW // C
G = C // L


def _body(z_hbm, src_hbm, dst_hbm, out_hbm,
          z_sh, idx_s0, idx_d0, idx_s1, idx_d1,
          rows_s0, rows_d0, rows_s1, rows_d1, out0, out1,
          sem_is0, sem_id0, sem_is1, sem_id1,
          sem_s0, sem_d0, sem_s1, sem_d1, sem_o0, sem_o1):
    cid = lax.axis_index("c")
    sid = lax.axis_index("s")
    wid = sid * NC + cid
    ebase = wid * EPW

    zrows = 1000

    @pl.when(sid < N_NODES // zrows)
    def _stage():
        pltpu.sync_copy(z_hbm.at[pl.ds(sid * zrows, zrows)],
                        z_sh.at[pl.ds(sid * zrows, zrows)])

    plsc.subcore_barrier()

    lanes = lax.iota(jnp.int32, L)
    bufs = ((idx_s0, idx_d0, sem_is0, sem_id0,
             rows_s0, rows_d0, sem_s0, sem_d0, out0, sem_o0),
            (idx_s1, idx_d1, sem_is1, sem_id1,
             rows_s1, rows_d1, sem_s1, sem_d1, out1, sem_o1))

    def start_idx(g, b):
        xs, xd, sis, sid_, _, _, _, _, _, _ = bufs[b]
        off = ebase + g * C
        pltpu.async_copy(src_hbm.at[pl.ds(off, C)], xs, sis)
        pltpu.async_copy(dst_hbm.at[pl.ds(off, C)], xd, sid_)

    def wait_idx(b):
        xs, xd, sis, sid_, _, _, _, _, _, _ = bufs[b]
        pltpu.make_async_copy(src_hbm.at[pl.ds(0, C)], xs, sis).wait()
        pltpu.make_async_copy(src_hbm.at[pl.ds(0, C)], xd, sid_).wait()

    def start_rows(b):
        xs, xd, _, _, rs, rd, ss, sd, _, _ = bufs[b]
        pltpu.async_copy(z_sh.at[xs], rs, ss)
        pltpu.async_copy(z_sh.at[xd], rd, sd)

    def wait_rows(b):
        _, _, _, _, rs, rd, ss, sd, _, _ = bufs[b]
        pltpu.make_async_copy(z_hbm.at[pl.ds(0, C)], rs, ss).wait()
        pltpu.make_async_copy(z_hbm.at[pl.ds(0, C)], rd, sd).wait()

    def compute(g, b):
        _, _, _, _, rs, rd, _, _, ob, so = bufs[b]

        @pl.when(g >= 2)
        def _drain():
            pltpu.make_async_copy(ob, out_hbm.at[pl.ds(0, C)], so).wait()

        def estep(blk, _):
            e_base = blk * L

            def dot16(e):
                p = [rs[e, pl.ds(j * L, L)] * rd[e, pl.ds(j * L, L)]
                     for j in range(D_FEAT // L)]
                while len(p) > 1:
                    p = [a + b for a, b in zip(p[::2], p[1::2])]
                return jnp.sum(p[0])

            def two_edges(u, res):
                e = e_base + u * 2
                res = jnp.where(lanes == u * 2, dot16(e), res)
                return jnp.where(lanes == u * 2 + 1, dot16(e + 1), res)

            res = lax.fori_loop(0, L // 2, two_edges,
                                jnp.zeros((L,), jnp.float32))
            ob[pl.ds(e_base, L)] = 1.0 / (1.0 + jnp.exp(-res))
            return _

        lax.fori_loop(0, G, estep, 0)
        pltpu.async_copy(ob, out_hbm.at[pl.ds(ebase + g * C, C)], so)

    start_idx(0, 0)
    wait_idx(0)
    start_rows(0)
    start_idx(1, 1)

    def pair(i, carry):
        g = i * 2
        wait_idx(1)
        start_rows(1)
        wait_rows(0)
        start_idx(g + 2, 0)
        compute(g, 0)
        wait_idx(0)
        start_rows(0)
        wait_rows(1)
        start_idx(jnp.minimum(g + 3, NCHUNK - 1), 1)
        compute(g + 1, 1)
        return carry

    lax.fori_loop(0, (NCHUNK - 1) // 2, pair, 0)
    wait_rows(0)
    compute(NCHUNK - 1, 0)
    wait_idx(1)

    pltpu.make_async_copy(out0, out_hbm.at[pl.ds(0, C)], sem_o0).wait()
    pltpu.make_async_copy(out1, out_hbm.at[pl.ds(0, C)], sem_o1).wait()


_mesh = plsc.VectorSubcoreMesh(
    core_axis_name="c", subcore_axis_name="s", num_cores=NC, num_subcores=NS)

_call = functools.partial(
    pl.kernel,
    out_type=jax.ShapeDtypeStruct((N_EDGES,), jnp.float32),
    mesh=_mesh,
    scratch_types=[
        pltpu.VMEM_SHARED((N_NODES, D_FEAT), jnp.float32),
        pltpu.VMEM((C,), jnp.int32),
        pltpu.VMEM((C,), jnp.int32),
        pltpu.VMEM((C,), jnp.int32),
        pltpu.VMEM((C,), jnp.int32),
        pltpu.VMEM((C, D_FEAT), jnp.float32),
        pltpu.VMEM((C, D_FEAT), jnp.float32),
        pltpu.VMEM((C, D_FEAT), jnp.float32),
        pltpu.VMEM((C, D_FEAT), jnp.float32),
        pltpu.VMEM((C,), jnp.float32),
        pltpu.VMEM((C,), jnp.float32),
        pltpu.SemaphoreType.DMA,
        pltpu.SemaphoreType.DMA,
        pltpu.SemaphoreType.DMA,
        pltpu.SemaphoreType.DMA,
        pltpu.SemaphoreType.DMA,
        pltpu.SemaphoreType.DMA,
        pltpu.SemaphoreType.DMA,
        pltpu.SemaphoreType.DMA,
        pltpu.SemaphoreType.DMA,
        pltpu.SemaphoreType.DMA,
    ],
    compiler_params=pltpu.CompilerParams(needs_layout_passes=False),
)(_body)


def kernel(z, edge_index):
    src = edge_index[0]
    dst = edge_index[1]
    return _call(z, src, dst)


# speedup vs baseline: 1.1563x; 1.1563x over previous
import functools

import jax
import jax.numpy as jnp
from jax import lax
from jax.experimental import pallas as pl
from jax.experimental.pallas import tpu as pltpu
from jax.experimental.pallas import tpu_sc as plsc

N_NODES = 10000
N_EDGES = 320000
D_FEAT = 128

NC = 2
NS = 16
L = 16
NW = NC * NS
EPW = N_EDGES // NW
C = 80
NCHUNK = EPW // C
G = C // L


def _body(z_hbm, src_hbm, dst_hbm, out_hbm,
          z_sh, idx_s0, idx_d0, idx_s1, idx_d1,
          rows_s0, rows_d0, rows_s1, rows_d1, out0, out1,
          sem_is0, sem_id0, sem_is1, sem_id1,
          sem_s0, sem_d0, sem_s1, sem_d1, sem_o0, sem_o1):
    cid = lax.axis_index("c")
    sid = lax.axis_index("s")
    wid = sid * NC + cid
    ebase = wid * EPW

    zrows = 2000

    @pl.when(sid < N_NODES // zrows)
    def _stage():
        pltpu.sync_copy(z_hbm.at[pl.ds(sid * zrows, zrows)],
                        z_sh.at[pl.ds(sid * zrows, zrows)])

    plsc.subcore_barrier()

    lanes = lax.iota(jnp.int32, L)
    bufs = ((idx_s0, idx_d0, sem_is0, sem_id0,
             rows_s0, rows_d0, sem_s0, sem_d0, out0, sem_o0),
            (idx_s1, idx_d1, sem_is1, sem_id1,
             rows_s1, rows_d1, sem_s1, sem_d1, out1, sem_o1))

    def start_idx(g, b):
        xs, xd, sis, sid_, _, _, _, _, _, _ = bufs[b]
        off = ebase + g * C
        pltpu.async_copy(src_hbm.at[pl.ds(off, C)], xs, sis)
        pltpu.async_copy(dst_hbm.at[pl.ds(off, C)], xd, sid_)

    def wait_idx(b):
        xs, xd, sis, sid_, _, _, _, _, _, _ = bufs[b]
        pltpu.make_async_copy(src_hbm.at[pl.ds(0, C)], xs, sis).wait()
        pltpu.make_async_copy(src_hbm.at[pl.ds(0, C)], xd, sid_).wait()

    def start_rows(b):
        xs, xd, _, _, rs, rd, ss, sd, _, _ = bufs[b]
        pltpu.async_copy(z_sh.at[xs], rs, ss)
        pltpu.async_copy(z_sh.at[xd], rd, sd)

    def wait_rows(b):
        _, _, _, _, rs, rd, ss, sd, _, _ = bufs[b]
        pltpu.make_async_copy(z_hbm.at[pl.ds(0, C)], rs, ss).wait()
        pltpu.make_async_copy(z_hbm.at[pl.ds(0, C)], rd, sd).wait()

    def compute(g, b):
        _, _, _, _, rs, rd, _, _, ob, so = bufs[b]

        @pl.when(g >= 2)
        def _drain():
            pltpu.make_async_copy(ob, out_hbm.at[pl.ds(0, C)], so).wait()

        def estep(blk, _):
            e_base = blk * L

            def dot16(e):
                p = []
                for j in range(D_FEAT // (2 * L)):
                    s2 = rs[e, pl.ds(j * 2 * L, 2 * L)]
                    d2 = rd[e, pl.ds(j * 2 * L, 2 * L)]
                    sa, sb = plsc.unpack(
                        s2, format=plsc.PackFormat.INTERLEAVED,
                        preferred_element_type=jnp.float32)
                    da, db = plsc.unpack(
                        d2, format=plsc.PackFormat.INTERLEAVED,
                        preferred_element_type=jnp.float32)
                    p.append(sa * da)
                    p.append(sb * db)
                while len(p) > 1:
                    p = [a + b for a, b in zip(p[::2], p[1::2])]
                return jnp.sum(p[0])

            def two_edges(u, res):
                e = e_base + u * 2
                res = jnp.where(lanes == u * 2, dot16(e), res)
                return jnp.where(lanes == u * 2 + 1, dot16(e + 1), res)

            res = lax.fori_loop(0, L // 2, two_edges,
                                jnp.zeros((L,), jnp.float32))
            ob[pl.ds(e_base, L)] = 1.0 / (1.0 + jnp.exp(-res))
            return _

        lax.fori_loop(0, G, estep, 0)
        pltpu.async_copy(ob, out_hbm.at[pl.ds(ebase + g * C, C)], so)

    start_idx(0, 0)
    wait_idx(0)
    start_rows(0)
    start_idx(1, 1)

    def pair(i, carry):
        g = i * 2
        wait_idx(1)
        start_rows(1)
        wait_rows(0)
        start_idx(g + 2, 0)
        compute(g, 0)
        wait_idx(0)
        start_rows(0)
        wait_rows(1)
        start_idx(jnp.minimum(g + 3, NCHUNK - 1), 1)
        compute(g + 1, 1)
        return carry

    lax.fori_loop(0, (NCHUNK - 1) // 2, pair, 0)
    wait_rows(0)
    compute(NCHUNK - 1, 0)
    wait_idx(1)

    pltpu.make_async_copy(out0, out_hbm.at[pl.ds(0, C)], sem_o0).wait()
    pltpu.make_async_copy(out1, out_hbm.at[pl.ds(0, C)], sem_o1).wait()


_mesh = plsc.VectorSubcoreMesh(
    core_axis_name="c", subcore_axis_name="s", num_cores=NC, num_subcores=NS)

_call = functools.partial(
    pl.kernel,
    out_type=jax.ShapeDtypeStruct((N_EDGES,), jnp.float32),
    mesh=_mesh,
    scratch_types=[
        pltpu.VMEM_SHARED((N_NODES, D_FEAT), jnp.bfloat16),
        pltpu.VMEM((C,), jnp.int32),
        pltpu.VMEM((C,), jnp.int32),
        pltpu.VMEM((C,), jnp.int32),
        pltpu.VMEM((C,), jnp.int32),
        pltpu.VMEM((C, D_FEAT), jnp.bfloat16),
        pltpu.VMEM((C, D_FEAT), jnp.bfloat16),
        pltpu.VMEM((C, D_FEAT), jnp.bfloat16),
        pltpu.VMEM((C, D_FEAT), jnp.bfloat16),
        pltpu.VMEM((C,), jnp.float32),
        pltpu.VMEM((C,), jnp.float32),
        pltpu.SemaphoreType.DMA,
        pltpu.SemaphoreType.DMA,
        pltpu.SemaphoreType.DMA,
        pltpu.SemaphoreType.DMA,
        pltpu.SemaphoreType.DMA,
        pltpu.SemaphoreType.DMA,
        pltpu.SemaphoreType.DMA,
        pltpu.SemaphoreType.DMA,
        pltpu.SemaphoreType.DMA,
        pltpu.SemaphoreType.DMA,
    ],
    compiler_params=pltpu.CompilerParams(needs_layout_passes=False,
                                         use_tc_tiling_on_sc=False),
)(_body)


def kernel(z, edge_index):
    src = edge_index[0]
    dst = edge_index[1]
    return _call(z.astype(jnp.bfloat16), src, dst)
